# baseline (speedup 1.0000x reference)
import functools

import jax
import jax.numpy as jnp
from jax import lax
from jax.experimental import pallas as pl
from jax.experimental.pallas import tpu as pltpu
from jax.experimental.pallas import tpu_sc as plsc

NC, NS = 2, 16
NW = NC * NS


K = 80
NB = 8
IF = 4


@functools.lru_cache(maxsize=None)
def _make_agg(n, h, e, with_cnt):
    epw = e // NW
    c = -(-epw // K)
    c = -(-c // NB) * NB
    na = n + 8
    npt = -(-(-(-na // NS)) // 8) * 8
    tail = na - (NS - 1) * npt
    tail_out = n - (NS - 1) * npt
    assert 0 < tail <= npt and 0 < tail_out

    mesh = plsc.VectorSubcoreMesh(
        core_axis_name="c", subcore_axis_name="s", num_cores=NC, num_subcores=NS
    )
    out_type = [jax.ShapeDtypeStruct((n, h), jnp.float32) for _ in range(2)]
    if with_cnt:
        out_type += [jax.ShapeDtypeStruct((n,), jnp.float32) for _ in range(2)]
    scratch = (
        [pltpu.VMEM((c, K), jnp.int32), pltpu.VMEM((c, K), jnp.int32)]
        + [pltpu.VMEM((K, h), jnp.float32) for _ in range(NB)]
        + ([pltpu.VMEM((K,), jnp.float32)] if with_cnt else [])
        + [pltpu.VMEM_SHARED((na, h), jnp.float32)]
        + ([pltpu.VMEM_SHARED((na,), jnp.float32)] if with_cnt else [])
        + [pltpu.SemaphoreType.DMA for _ in range(2 * NB)]
    )
    jhi = c // NB

    def body(*refs):
        if with_cnt:
            (p_hbm, srcg, dstg, z2, z1, ones_h, agg_a, agg_b, cnt_a, cnt_b,
             src_v, dst_v, *rest) = refs
            rows = rest[:NB]
            ones_v, acc, cnt = rest[NB], rest[NB + 1], rest[NB + 2]
            gsem = rest[NB + 3:NB + 3 + NB]
            ssem = rest[NB + 3 + NB:]
        else:
            p_hbm, srcg, dstg, z2, agg_a, agg_b, src_v, dst_v, *rest = refs
            rows = rest[:NB]
            acc = rest[NB]
            gsem = rest[NB + 1:NB + 1 + NB]
            ssem = rest[NB + 1 + NB:]

        cid = lax.axis_index("c")
        sid = lax.axis_index("s")
        wid = sid * NC + cid

        pltpu.sync_copy(srcg.at[wid], src_v)
        pltpu.sync_copy(dstg.at[wid], dst_v)

        @pl.when(sid < NS - 1)
        def _():
            pltpu.sync_copy(z2, acc.at[pl.ds(sid * npt, npt)])

        @pl.when(sid == NS - 1)
        def _():
            pltpu.sync_copy(z2.at[pl.ds(0, tail)],
                            acc.at[pl.ds((NS - 1) * npt, tail)])

        if with_cnt:
            pltpu.sync_copy(ones_h, ones_v)

            @pl.when(sid == 0)
            def _():
                pltpu.sync_copy(z1, cnt)

        plsc.subcore_barrier()

        def drain_scatter(bn):
            pltpu.make_async_copy(rows[bn], acc.at[dst_v.at[0]], ssem[bn]).wait()
            if with_cnt:
                pltpu.make_async_copy(ones_v, cnt.at[dst_v.at[0]], ssem[bn]).wait()

        for b in range(IF):
            pltpu.async_copy(p_hbm.at[src_v.at[b]], rows[b], gsem[b])

        def step(j, carry):
            for b in range(NB):
                ch = j * NB + b
                pltpu.make_async_copy(p_hbm.at[src_v.at[0]], rows[b], gsem[b]).wait()
                pltpu.async_copy(rows[b], acc.at[dst_v.at[ch]], ssem[b], add=True)
                if with_cnt:
                    pltpu.async_copy(ones_v, cnt.at[dst_v.at[ch]], ssem[b], add=True)
                bn = (b + IF) % NB
                nxt = ch + IF

                def fire(bn=bn, nxt=nxt):
                    pltpu.async_copy(p_hbm.at[src_v.at[nxt]], rows[bn], gsem[bn])

                if b < IF:
                    pl.when(j >= 1)(functools.partial(drain_scatter, bn))
                    fire()
                else:
                    drain_scatter(bn)
                    pl.when(j <= jhi - 2)(fire)
            return carry

        lax.fori_loop(0, jhi, step, 0)
        for b in range(IF, NB):
            drain_scatter(b)
        plsc.subcore_barrier()

        for my_cid, agg_out in ((0, agg_a), (1, agg_b)):
            @pl.when((cid == my_cid) & (sid < NS - 1))
            def _(agg_out=agg_out):
                slc = pl.ds(sid * npt, npt)
                pltpu.sync_copy(acc.at[slc], agg_out.at[slc])

            @pl.when((cid == my_cid) & (sid == NS - 1))
            def _(agg_out=agg_out):
                slc = pl.ds((NS - 1) * npt, tail_out)
                pltpu.sync_copy(acc.at[slc], agg_out.at[slc])

        if with_cnt:
            @pl.when((cid == 0) & (sid == 0))
            def _():
                pltpu.sync_copy(cnt.at[pl.ds(0, n)], cnt_a)

            @pl.when((cid == 1) & (sid == 0))
            def _():
                pltpu.sync_copy(cnt.at[pl.ds(0, n)], cnt_b)

    return pl.kernel(
        body, out_type=out_type, mesh=mesh, scratch_types=scratch,
        compiler_params=pltpu.CompilerParams(use_tc_tiling_on_sc=False),
    )


def _proj2_body(x_ref, wl_ref, wr_ref, p_ref, r_ref):
    xb = x_ref[...]
    p_ref[...] = jnp.dot(xb, wl_ref[...], preferred_element_type=jnp.float32)
    r_ref[...] = jnp.dot(xb, wr_ref[...], preferred_element_type=jnp.float32)


def _proj2(x, wl, wr, bn=1000):
    n, d = x.shape
    h = wl.shape[1]
    return pl.pallas_call(
        _proj2_body,
        grid=(n // bn,),
        in_specs=[
            pl.BlockSpec((bn, d), lambda i: (i, 0)),
            pl.BlockSpec((d, h), lambda i: (0, 0)),
            pl.BlockSpec((d, h), lambda i: (0, 0)),
        ],
        out_specs=[pl.BlockSpec((bn, h), lambda i: (i, 0))] * 2,
        out_shape=[jax.ShapeDtypeStruct((n, h), jnp.float32)] * 2,
    )(x, wl, wr)


def _comb_body(a0, a1, c0, c1, r, bl, wl, wr, p_out, r_out):
    cnt = jnp.maximum(c0[...] + c1[...], 1.0)
    hh = jnp.maximum((a0[...] + a1[...]) / cnt + bl[...] + r[...], 0.0)
    p_out[...] = jnp.dot(hh, wl[...], preferred_element_type=jnp.float32)
    r_out[...] = jnp.dot(hh, wr[...], preferred_element_type=jnp.float32)


def _comb(a0, a1, c0, c1, r, bl, wl, wr, bn=1000):
    n, h = a0.shape
    row = pl.BlockSpec((bn, h), lambda i: (i, 0))
    one = pl.BlockSpec((bn, 1), lambda i: (i, 0))
    full = pl.BlockSpec((h, h), lambda i: (0, 0))
    return pl.pallas_call(
        _comb_body,
        grid=(n // bn,),
        in_specs=[row, row, one, one, row,
                  pl.BlockSpec((1, h), lambda i: (0, 0)), full, full],
        out_specs=[row] * 2,
        out_shape=[jax.ShapeDtypeStruct((n, h), jnp.float32)] * 2,
    )(a0, a1, c0, c1, r, bl, wl, wr)


def _fin_body(a0, a1, c0, c1, r, bl, wc, bc, o_ref):
    cnt = jnp.maximum(c0[...] + c1[...], 1.0)
    hh = jnp.maximum((a0[...] + a1[...]) / cnt + bl[...] + r[...], 0.0)
    o_ref[...] = jnp.dot(hh, wc[...], preferred_element_type=jnp.float32) + bc[...]


def _fin(a0, a1, c0, c1, r, bl, wc, bc, bn=1000):
    n, h = a0.shape
    row = pl.BlockSpec((bn, h), lambda i: (i, 0))
    one = pl.BlockSpec((bn, 1), lambda i: (i, 0))
    return pl.pallas_call(
        _fin_body,
        grid=(n // bn,),
        in_specs=[row, row, one, one, row,
                  pl.BlockSpec((1, h), lambda i: (0, 0)),
                  pl.BlockSpec((h, 1), lambda i: (0, 0)),
                  pl.BlockSpec((1, 1), lambda i: (0, 0))],
        out_specs=pl.BlockSpec((bn, 1), lambda i: (i, 0)),
        out_shape=jax.ShapeDtypeStruct((n, 1), jnp.float32),
    )(a0, a1, c0, c1, r, bl, wc, bc)


def kernel(x, edge_index, Wl1, bl1, Wr1, Wl2, bl2, Wr2, Wc, bc):
    n, d = x.shape
    h = Wl1.shape[1]
    e = edge_index.shape[1]
    epw = e // NW
    c = -(-(-(-epw // K)) // NB) * NB
    pad = c * K - epw
    na = n + 8
    npt = -(-(-(-na // NS)) // 8) * 8

    src = jnp.pad(edge_index[0].reshape(NW, epw), ((0, 0), (0, pad)))
    src = src.reshape(NW, c, K)
    dst = jnp.pad(edge_index[1].reshape(NW, epw), ((0, 0), (0, pad)),
                  constant_values=n)
    dst = dst.reshape(NW, c, K)
    zeros2 = jnp.zeros((npt, h), jnp.float32)
    zeros1 = jnp.zeros((na,), jnp.float32)
    ones = jnp.ones((K,), jnp.float32)

    p1, r1 = _proj2(x, Wl1, Wr1)
    a0, a1, c0, c1 = _make_agg(n, h, e, True)(p1, src, dst, zeros2, zeros1, ones)
    c0 = c0.reshape(n, 1)
    c1 = c1.reshape(n, 1)
    p2, r2 = _comb(a0, a1, c0, c1, r1, bl1.reshape(1, h), Wl2, Wr2)
    b0, b1 = _make_agg(n, h, e, False)(p2, src, dst, zeros2)
    return _fin(b0, b1, c0, c1, r2, bl2.reshape(1, h), Wc, bc.reshape(1, 1))


# speedup vs baseline: 1.0222x; 1.0222x over previous
import functools

import jax
import jax.numpy as jnp
from jax import lax
from jax.experimental import pallas as pl
from jax.experimental.pallas import tpu as pltpu
from jax.experimental.pallas import tpu_sc as plsc

NC, NS = 2, 16
NW = NC * NS


K = 128
NB = 8


@functools.lru_cache(maxsize=None)
def _make_agg(n, h, e, with_cnt):
    epw = e // NW
    c = -(-epw // K)
    c = -(-c // NB) * NB
    na = n + 8
    npt = -(-(-(-na // NS)) // 8) * 8
    tail = na - (NS - 1) * npt
    tail_out = n - (NS - 1) * npt
    assert 0 < tail <= npt and 0 < tail_out

    mesh = plsc.VectorSubcoreMesh(
        core_axis_name="c", subcore_axis_name="s", num_cores=NC, num_subcores=NS
    )
    out_type = [jax.ShapeDtypeStruct((n, h), jnp.float32) for _ in range(2)]
    if with_cnt:
        out_type += [jax.ShapeDtypeStruct((n,), jnp.float32) for _ in range(2)]
    scratch = (
        [pltpu.VMEM((c, K), jnp.int32), pltpu.VMEM((c, K), jnp.int32)]
        + [pltpu.VMEM((K, h), jnp.float32) for _ in range(NB)]
        + ([pltpu.VMEM((K,), jnp.float32)] if with_cnt else [])
        + [pltpu.VMEM_SHARED((na, h), jnp.float32)]
        + ([pltpu.VMEM_SHARED((na,), jnp.float32)] if with_cnt else [])
        + [pltpu.SemaphoreType.DMA for _ in range(NB)]
    )
    jhi = c // NB

    def body(*refs):
        if with_cnt:
            (p_hbm, srcg, dstg, z2, z1, ones_h, agg_a, agg_b, cnt_a, cnt_b,
             src_v, dst_v, *rest) = refs
            rows = rest[:NB]
            ones_v, acc, cnt = rest[NB], rest[NB + 1], rest[NB + 2]
            gsem = rest[NB + 3:]
        else:
            p_hbm, srcg, dstg, z2, agg_a, agg_b, src_v, dst_v, *rest = refs
            rows = rest[:NB]
            acc = rest[NB]
            gsem = rest[NB + 1:]

        cid = lax.axis_index("c")
        sid = lax.axis_index("s")
        wid = sid * NC + cid

        pltpu.sync_copy(srcg.at[wid], src_v)
        pltpu.sync_copy(dstg.at[wid], dst_v)

        @pl.when(sid < NS - 1)
        def _():
            pltpu.sync_copy(z2, acc.at[pl.ds(sid * npt, npt)])

        @pl.when(sid == NS - 1)
        def _():
            pltpu.sync_copy(z2.at[pl.ds(0, tail)],
                            acc.at[pl.ds((NS - 1) * npt, tail)])

        if with_cnt:
            pltpu.sync_copy(ones_h, ones_v)

            @pl.when(sid == 0)
            def _():
                pltpu.sync_copy(z1, cnt)

        plsc.subcore_barrier()

        for b in range(NB):
            pltpu.async_copy(p_hbm.at[src_v.at[b]], rows[b], gsem[b])

        def step(j, carry):
            for b in range(NB):
                ch = j * NB + b
                pltpu.make_async_copy(p_hbm.at[src_v.at[0]], rows[b], gsem[b]).wait()
                pltpu.sync_copy(rows[b], acc.at[dst_v.at[ch]], add=True)
                if with_cnt:
                    pltpu.sync_copy(ones_v, cnt.at[dst_v.at[ch]], add=True)

                def fire(b=b, nxt=ch + NB):
                    pltpu.async_copy(p_hbm.at[src_v.at[nxt]], rows[b], gsem[b])

                pl.when(j <= jhi - 2)(fire)
            return carry

        lax.fori_loop(0, jhi, step, 0)
        plsc.subcore_barrier()

        for my_cid, agg_out in ((0, agg_a), (1, agg_b)):
            @pl.when((cid == my_cid) & (sid < NS - 1))
            def _(agg_out=agg_out):
                slc = pl.ds(sid * npt, npt)
                pltpu.sync_copy(acc.at[slc], agg_out.at[slc])

            @pl.when((cid == my_cid) & (sid == NS - 1))
            def _(agg_out=agg_out):
                slc = pl.ds((NS - 1) * npt, tail_out)
                pltpu.sync_copy(acc.at[slc], agg_out.at[slc])

        if with_cnt:
            @pl.when((cid == 0) & (sid == 0))
            def _():
                pltpu.sync_copy(cnt.at[pl.ds(0, n)], cnt_a)

            @pl.when((cid == 1) & (sid == 0))
            def _():
                pltpu.sync_copy(cnt.at[pl.ds(0, n)], cnt_b)

    return pl.kernel(
        body, out_type=out_type, mesh=mesh, scratch_types=scratch,
        compiler_params=pltpu.CompilerParams(use_tc_tiling_on_sc=False),
    )


def _proj2_body(x_ref, wl_ref, wr_ref, p_ref, r_ref):
    xb = x_ref[...]
    p_ref[...] = jnp.dot(xb, wl_ref[...], preferred_element_type=jnp.float32)
    r_ref[...] = jnp.dot(xb, wr_ref[...], preferred_element_type=jnp.float32)


def _proj2(x, wl, wr, bn=1000):
    n, d = x.shape
    h = wl.shape[1]
    return pl.pallas_call(
        _proj2_body,
        grid=(n // bn,),
        in_specs=[
            pl.BlockSpec((bn, d), lambda i: (i, 0)),
            pl.BlockSpec((d, h), lambda i: (0, 0)),
            pl.BlockSpec((d, h), lambda i: (0, 0)),
        ],
        out_specs=[pl.BlockSpec((bn, h), lambda i: (i, 0))] * 2,
        out_shape=[jax.ShapeDtypeStruct((n, h), jnp.float32)] * 2,
    )(x, wl, wr)


def _comb_body(a0, a1, c0, c1, r, bl, wl, wr, p_out, r_out):
    cnt = jnp.maximum(c0[...] + c1[...], 1.0)
    hh = jnp.maximum((a0[...] + a1[...]) / cnt + bl[...] + r[...], 0.0)
    p_out[...] = jnp.dot(hh, wl[...], preferred_element_type=jnp.float32)
    r_out[...] = jnp.dot(hh, wr[...], preferred_element_type=jnp.float32)


def _comb(a0, a1, c0, c1, r, bl, wl, wr, bn=1000):
    n, h = a0.shape
    row = pl.BlockSpec((bn, h), lambda i: (i, 0))
    one = pl.BlockSpec((bn, 1), lambda i: (i, 0))
    full = pl.BlockSpec((h, h), lambda i: (0, 0))
    return pl.pallas_call(
        _comb_body,
        grid=(n // bn,),
        in_specs=[row, row, one, one, row,
                  pl.BlockSpec((1, h), lambda i: (0, 0)), full, full],
        out_specs=[row] * 2,
        out_shape=[jax.ShapeDtypeStruct((n, h), jnp.float32)] * 2,
    )(a0, a1, c0, c1, r, bl, wl, wr)


def _fin_body(a0, a1, c0, c1, r, bl, wc, bc, o_ref):
    cnt = jnp.maximum(c0[...] + c1[...], 1.0)
    hh = jnp.maximum((a0[...] + a1[...]) / cnt + bl[...] + r[...], 0.0)
    o_ref[...] = jnp.dot(hh, wc[...], preferred_element_type=jnp.float32) + bc[...]


def _fin(a0, a1, c0, c1, r, bl, wc, bc, bn=1000):
    n, h = a0.shape
    row = pl.BlockSpec((bn, h), lambda i: (i, 0))
    one = pl.BlockSpec((bn, 1), lambda i: (i, 0))
    return pl.pallas_call(
        _fin_body,
        grid=(n // bn,),
        in_specs=[row, row, one, one, row,
                  pl.BlockSpec((1, h), lambda i: (0, 0)),
                  pl.BlockSpec((h, 1), lambda i: (0, 0)),
                  pl.BlockSpec((1, 1), lambda i: (0, 0))],
        out_specs=pl.BlockSpec((bn, 1), lambda i: (i, 0)),
        out_shape=jax.ShapeDtypeStruct((n, 1), jnp.float32),
    )(a0, a1, c0, c1, r, bl, wc, bc)


def kernel(x, edge_index, Wl1, bl1, Wr1, Wl2, bl2, Wr2, Wc, bc):
    n, d = x.shape
    h = Wl1.shape[1]
    e = edge_index.shape[1]
    epw = e // NW
    c = -(-(-(-epw // K)) // NB) * NB
    pad = c * K - epw
    na = n + 8
    npt = -(-(-(-na // NS)) // 8) * 8

    src = jnp.pad(edge_index[0].reshape(NW, epw), ((0, 0), (0, pad)))
    src = src.reshape(NW, c, K)
    dst = jnp.pad(edge_index[1].reshape(NW, epw), ((0, 0), (0, pad)),
                  constant_values=n)
    dst = dst.reshape(NW, c, K)
    zeros2 = jnp.zeros((npt, h), jnp.float32)
    zeros1 = jnp.zeros((na,), jnp.float32)
    ones = jnp.ones((K,), jnp.float32)

    p1, r1 = _proj2(x, Wl1, Wr1)
    a0, a1, c0, c1 = _make_agg(n, h, e, True)(p1, src, dst, zeros2, zeros1, ones)
    c0 = c0.reshape(n, 1)
    c1 = c1.reshape(n, 1)
    p2, r2 = _comb(a0, a1, c0, c1, r1, bl1.reshape(1, h), Wl2, Wr2)
    b0, b1 = _make_agg(n, h, e, False)(p2, src, dst, zeros2)
    return _fin(b0, b1, c0, c1, r2, bl2.reshape(1, h), Wc, bc.reshape(1, 1))


# speedup vs baseline: 1.0224x; 1.0003x over previous
import functools

import jax
import jax.numpy as jnp
from jax import lax
from jax.experimental import pallas as pl
from jax.experimental.pallas import tpu as pltpu
from jax.experimental.pallas import tpu_sc as plsc

NC, NS = 2, 16
NW = NC * NS


K = 128
NB = 8
PADR = 128


def _acc_geom(n):
    na = n + PADR
    npt = -(-(-(-na // NS)) // 8) * 8
    tail = na - (NS - 1) * npt
    tail_out = n - (NS - 1) * npt
    assert 0 < tail <= npt and 0 < tail_out
    return na, npt, tail, tail_out


@functools.lru_cache(maxsize=None)
def _make_agg(n, h, e, with_cnt):
    epw = e // NW
    c = -(-epw // K)
    c = -(-c // NB) * NB
    na, npt, tail, tail_out = _acc_geom(n)

    mesh = plsc.VectorSubcoreMesh(
        core_axis_name="c", subcore_axis_name="s", num_cores=NC, num_subcores=NS
    )
    out_type = [jax.ShapeDtypeStruct((n, h), jnp.float32) for _ in range(2)]
    if with_cnt:
        out_type += [jax.ShapeDtypeStruct((n,), jnp.float32) for _ in range(2)]
    scratch = (
        [pltpu.VMEM((c, K), jnp.int32), pltpu.VMEM((c, K), jnp.int32)]
        + [pltpu.VMEM((K, h), jnp.float32) for _ in range(NB)]
        + ([pltpu.VMEM((K,), jnp.float32)] if with_cnt else [])
        + [pltpu.VMEM_SHARED((na, h), jnp.float32)]
        + ([pltpu.VMEM_SHARED((na,), jnp.float32)] if with_cnt else [])
        + [pltpu.SemaphoreType.DMA for _ in range(NB)]
    )
    jhi = c // NB

    def body(*refs):
        if with_cnt:
            (p_hbm, srcg, dstg, z2, z1, ones_h, agg_a, agg_b, cnt_a, cnt_b,
             src_v, dst_v, *rest) = refs
            rows = rest[:NB]
            ones_v, acc, cnt = rest[NB], rest[NB + 1], rest[NB + 2]
            gsem = rest[NB + 3:]
        else:
            p_hbm, srcg, dstg, z2, agg_a, agg_b, src_v, dst_v, *rest = refs
            rows = rest[:NB]
            acc = rest[NB]
            gsem = rest[NB + 1:]

        cid = lax.axis_index("c")
        sid = lax.axis_index("s")
        wid = sid * NC + cid

        pltpu.sync_copy(srcg.at[wid], src_v)
        pltpu.sync_copy(dstg.at[wid], dst_v)

        @pl.when(sid < NS - 1)
        def _():
            pltpu.sync_copy(z2, acc.at[pl.ds(sid * npt, npt)])

        @pl.when(sid == NS - 1)
        def _():
            pltpu.sync_copy(z2.at[pl.ds(0, tail)],
                            acc.at[pl.ds((NS - 1) * npt, tail)])

        if with_cnt:
            pltpu.sync_copy(ones_h, ones_v)

            @pl.when(sid == 0)
            def _():
                pltpu.sync_copy(z1, cnt)

        plsc.subcore_barrier()

        for b in range(NB):
            pltpu.async_copy(p_hbm.at[src_v.at[b]], rows[b], gsem[b])

        def step(j, carry):
            for b in range(NB):
                ch = j * NB + b
                pltpu.make_async_copy(p_hbm.at[src_v.at[0]], rows[b], gsem[b]).wait()
                pltpu.sync_copy(rows[b], acc.at[dst_v.at[ch]], add=True)
                if with_cnt:
                    pltpu.sync_copy(ones_v, cnt.at[dst_v.at[ch]], add=True)

                def fire(b=b, nxt=ch + NB):
                    pltpu.async_copy(p_hbm.at[src_v.at[nxt]], rows[b], gsem[b])

                pl.when(j <= jhi - 2)(fire)
            return carry

        lax.fori_loop(0, jhi, step, 0)
        plsc.subcore_barrier()

        for my_cid, agg_out in ((0, agg_a), (1, agg_b)):
            @pl.when((cid == my_cid) & (sid < NS - 1))
            def _(agg_out=agg_out):
                slc = pl.ds(sid * npt, npt)
                pltpu.sync_copy(acc.at[slc], agg_out.at[slc])

            @pl.when((cid == my_cid) & (sid == NS - 1))
            def _(agg_out=agg_out):
                slc = pl.ds((NS - 1) * npt, tail_out)
                pltpu.sync_copy(acc.at[slc], agg_out.at[slc])

        if with_cnt:
            @pl.when((cid == 0) & (sid == 0))
            def _():
                pltpu.sync_copy(cnt.at[pl.ds(0, n)], cnt_a)

            @pl.when((cid == 1) & (sid == 0))
            def _():
                pltpu.sync_copy(cnt.at[pl.ds(0, n)], cnt_b)

    return pl.kernel(
        body, out_type=out_type, mesh=mesh, scratch_types=scratch,
        compiler_params=pltpu.CompilerParams(use_tc_tiling_on_sc=False),
    )


def _proj2_body(x_ref, wl_ref, wr_ref, p_ref, r_ref):
    xb = x_ref[...]
    p_ref[...] = jnp.dot(xb, wl_ref[...], preferred_element_type=jnp.float32)
    r_ref[...] = jnp.dot(xb, wr_ref[...], preferred_element_type=jnp.float32)


def _proj2(x, wl, wr, bn=1000):
    n, d = x.shape
    h = wl.shape[1]
    return pl.pallas_call(
        _proj2_body,
        grid=(n // bn,),
        in_specs=[
            pl.BlockSpec((bn, d), lambda i: (i, 0)),
            pl.BlockSpec((d, h), lambda i: (0, 0)),
            pl.BlockSpec((d, h), lambda i: (0, 0)),
        ],
        out_specs=[pl.BlockSpec((bn, h), lambda i: (i, 0))] * 2,
        out_shape=[jax.ShapeDtypeStruct((n, h), jnp.float32)] * 2,
    )(x, wl, wr)


def _comb_body(a0, a1, c0, c1, r, bl, wl, wr, p_out, r_out):
    cnt = jnp.maximum(c0[...] + c1[...], 1.0)
    hh = jnp.maximum((a0[...] + a1[...]) / cnt + bl[...] + r[...], 0.0)
    p_out[...] = jnp.dot(hh, wl[...], preferred_element_type=jnp.float32)
    r_out[...] = jnp.dot(hh, wr[...], preferred_element_type=jnp.float32)


def _comb(a0, a1, c0, c1, r, bl, wl, wr, bn=1000):
    n, h = a0.shape
    row = pl.BlockSpec((bn, h), lambda i: (i, 0))
    one = pl.BlockSpec((bn, 1), lambda i: (i, 0))
    full = pl.BlockSpec((h, h), lambda i: (0, 0))
    return pl.pallas_call(
        _comb_body,
        grid=(n // bn,),
        in_specs=[row, row, one, one, row,
                  pl.BlockSpec((1, h), lambda i: (0, 0)), full, full],
        out_specs=[row] * 2,
        out_shape=[jax.ShapeDtypeStruct((n, h), jnp.float32)] * 2,
    )(a0, a1, c0, c1, r, bl, wl, wr)


def _fin_body(a0, a1, c0, c1, r, bl, wc, bc, o_ref):
    cnt = jnp.maximum(c0[...] + c1[...], 1.0)
    hh = jnp.maximum((a0[...] + a1[...]) / cnt + bl[...] + r[...], 0.0)
    o_ref[...] = jnp.dot(hh, wc[...], preferred_element_type=jnp.float32) + bc[...]


def _fin(a0, a1, c0, c1, r, bl, wc, bc, bn=1000):
    n, h = a0.shape
    row = pl.BlockSpec((bn, h), lambda i: (i, 0))
    one = pl.BlockSpec((bn, 1), lambda i: (i, 0))
    return pl.pallas_call(
        _fin_body,
        grid=(n // bn,),
        in_specs=[row, row, one, one, row,
                  pl.BlockSpec((1, h), lambda i: (0, 0)),
                  pl.BlockSpec((h, 1), lambda i: (0, 0)),
                  pl.BlockSpec((1, 1), lambda i: (0, 0))],
        out_specs=pl.BlockSpec((bn, 1), lambda i: (i, 0)),
        out_shape=jax.ShapeDtypeStruct((n, 1), jnp.float32),
    )(a0, a1, c0, c1, r, bl, wc, bc)


def kernel(x, edge_index, Wl1, bl1, Wr1, Wl2, bl2, Wr2, Wc, bc):
    n, d = x.shape
    h = Wl1.shape[1]
    e = edge_index.shape[1]
    epw = e // NW
    c = -(-(-(-epw // K)) // NB) * NB
    pad = c * K - epw
    na, npt, _, _ = _acc_geom(n)

    src = jnp.pad(edge_index[0].reshape(NW, epw), ((0, 0), (0, pad)))
    src = src.reshape(NW, c, K)
    padvals = n + (jnp.arange(pad, dtype=jnp.int32) % PADR)
    padvals = jnp.broadcast_to(padvals, (NW, pad))
    dst = jnp.concatenate([edge_index[1].reshape(NW, epw), padvals], axis=1)
    dst = dst.reshape(NW, c, K)
    zeros2 = jnp.zeros((npt, h), jnp.float32)
    zeros1 = jnp.zeros((na,), jnp.float32)
    ones = jnp.ones((K,), jnp.float32)

    p1, r1 = _proj2(x, Wl1, Wr1)
    a0, a1, c0, c1 = _make_agg(n, h, e, True)(p1, src, dst, zeros2, zeros1, ones)
    c0 = c0.reshape(n, 1)
    c1 = c1.reshape(n, 1)
    p2, r2 = _comb(a0, a1, c0, c1, r1, bl1.reshape(1, h), Wl2, Wr2)
    b0, b1 = _make_agg(n, h, e, False)(p2, src, dst, zeros2)
    return _fin(b0, b1, c0, c1, r2, bl2.reshape(1, h), Wc, bc.reshape(1, 1))


# speedup vs baseline: 2.4178x; 2.3648x over previous
import functools

import jax
import jax.numpy as jnp
from jax import lax
from jax.experimental import pallas as pl
from jax.experimental.pallas import tpu as pltpu
from jax.experimental.pallas import tpu_sc as plsc

NC, NS = 2, 16
NW = NC * NS


K = 80
NB = 5
PADR = 128


def _acc_geom(n):
    na = n + PADR
    npt = -(-(-(-na // NS)) // 8) * 8
    tail = na - (NS - 1) * npt
    tail_out = n - (NS - 1) * npt
    assert 0 < tail <= npt and 0 < tail_out
    return na, npt, tail, tail_out


@functools.lru_cache(maxsize=None)
def _make_agg(n, h, e, with_cnt):
    epw = e // NW
    c = -(-epw // K)
    c = -(-c // NB) * NB
    na, npt, tail, tail_out = _acc_geom(n)

    mesh = plsc.VectorSubcoreMesh(
        core_axis_name="c", subcore_axis_name="s", num_cores=NC, num_subcores=NS
    )
    out_type = [jax.ShapeDtypeStruct((n, h), jnp.float32) for _ in range(2)]
    if with_cnt:
        out_type += [jax.ShapeDtypeStruct((n,), jnp.float32) for _ in range(2)]
    scratch = (
        [pltpu.VMEM((c, K), jnp.int32), pltpu.VMEM((c, K), jnp.int32)]
        + [pltpu.VMEM((K, h), jnp.float32) for _ in range(NB)]
        + ([pltpu.VMEM((K,), jnp.float32)] if with_cnt else [])
        + [pltpu.VMEM_SHARED((na, h), jnp.float32)]
        + ([pltpu.VMEM_SHARED((na,), jnp.float32)] if with_cnt else [])
        + [pltpu.SemaphoreType.DMA for _ in range(NB)]
    )
    jhi = c // NB

    def body(*refs):
        if with_cnt:
            (p_hbm, srcg, dstg, z2, z1, ones_h, agg_a, agg_b, cnt_a, cnt_b,
             src_v, dst_v, *rest) = refs
            rows = rest[:NB]
            ones_v, acc, cnt = rest[NB], rest[NB + 1], rest[NB + 2]
            gsem = rest[NB + 3:]
        else:
            p_hbm, srcg, dstg, z2, agg_a, agg_b, src_v, dst_v, *rest = refs
            rows = rest[:NB]
            acc = rest[NB]
            gsem = rest[NB + 1:]

        cid = lax.axis_index("c")
        sid = lax.axis_index("s")
        wid = sid * NC + cid

        pltpu.sync_copy(srcg.at[wid], src_v)
        pltpu.sync_copy(dstg.at[wid], dst_v)

        @pl.when(sid < NS - 1)
        def _():
            pltpu.sync_copy(z2, acc.at[pl.ds(sid * npt, npt)])

        @pl.when(sid == NS - 1)
        def _():
            pltpu.sync_copy(z2.at[pl.ds(0, tail)],
                            acc.at[pl.ds((NS - 1) * npt, tail)])

        if with_cnt:
            pltpu.sync_copy(ones_h, ones_v)

            @pl.when(sid == 0)
            def _():
                pltpu.sync_copy(z1, cnt)

        plsc.subcore_barrier()

        for b in range(NB):
            pltpu.async_copy(p_hbm.at[src_v.at[b]], rows[b], gsem[b])

        def step(j, carry):
            for b in range(NB):
                ch = j * NB + b
                pltpu.make_async_copy(p_hbm.at[src_v.at[0]], rows[b], gsem[b]).wait()
                pltpu.sync_copy(rows[b], acc.at[dst_v.at[ch]], add=True)
                if with_cnt:
                    pltpu.sync_copy(ones_v, cnt.at[dst_v.at[ch]], add=True)

                def fire(b=b, nxt=ch + NB):
                    pltpu.async_copy(p_hbm.at[src_v.at[nxt]], rows[b], gsem[b])

                pl.when(j <= jhi - 2)(fire)
            return carry

        lax.fori_loop(0, jhi, step, 0)
        plsc.subcore_barrier()

        for my_cid, agg_out in ((0, agg_a), (1, agg_b)):
            @pl.when((cid == my_cid) & (sid < NS - 1))
            def _(agg_out=agg_out):
                slc = pl.ds(sid * npt, npt)
                pltpu.sync_copy(acc.at[slc], agg_out.at[slc])

            @pl.when((cid == my_cid) & (sid == NS - 1))
            def _(agg_out=agg_out):
                slc = pl.ds((NS - 1) * npt, tail_out)
                pltpu.sync_copy(acc.at[slc], agg_out.at[slc])

        if with_cnt:
            @pl.when((cid == 0) & (sid == 0))
            def _():
                pltpu.sync_copy(cnt.at[pl.ds(0, n)], cnt_a)

            @pl.when((cid == 1) & (sid == 0))
            def _():
                pltpu.sync_copy(cnt.at[pl.ds(0, n)], cnt_b)

    return pl.kernel(
        body, out_type=out_type, mesh=mesh, scratch_types=scratch,
        compiler_params=pltpu.CompilerParams(use_tc_tiling_on_sc=False),
    )


def _proj2_body(x_ref, wl_ref, wr_ref, p_ref, r_ref):
    xb = x_ref[...]
    p_ref[...] = jnp.dot(xb, wl_ref[...], preferred_element_type=jnp.float32)
    r_ref[...] = jnp.dot(xb, wr_ref[...], preferred_element_type=jnp.float32)


def _proj2(x, wl, wr, bn=2000):
    n, d = x.shape
    h = wl.shape[1]
    return pl.pallas_call(
        _proj2_body,
        grid=(n // bn,),
        in_specs=[
            pl.BlockSpec((bn, d), lambda i: (i, 0)),
            pl.BlockSpec((d, h), lambda i: (0, 0)),
            pl.BlockSpec((d, h), lambda i: (0, 0)),
        ],
        out_specs=[pl.BlockSpec((bn, h), lambda i: (i, 0))] * 2,
        out_shape=[jax.ShapeDtypeStruct((n, h), jnp.float32)] * 2,
    )(x, wl, wr)


def _comb_body(a0, a1, c0, c1, r, bl, wl, wr, p_out, r_out):
    cnt = jnp.maximum(c0[...] + c1[...], 1.0)
    hh = jnp.maximum((a0[...] + a1[...]) / cnt + bl[...] + r[...], 0.0)
    p_out[...] = jnp.dot(hh, wl[...], preferred_element_type=jnp.float32)
    r_out[...] = jnp.dot(hh, wr[...], preferred_element_type=jnp.float32)


def _comb(a0, a1, c0, c1, r, bl, wl, wr, bn=2000):
    n, h = a0.shape
    row = pl.BlockSpec((bn, h), lambda i: (i, 0))
    one = pl.BlockSpec((bn, 1), lambda i: (i, 0))
    full = pl.BlockSpec((h, h), lambda i: (0, 0))
    return pl.pallas_call(
        _comb_body,
        grid=(n // bn,),
        in_specs=[row, row, one, one, row,
                  pl.BlockSpec((1, h), lambda i: (0, 0)), full, full],
        out_specs=[row] * 2,
        out_shape=[jax.ShapeDtypeStruct((n, h), jnp.float32)] * 2,
    )(a0, a1, c0, c1, r, bl, wl, wr)


def _fin_body(a0, a1, c0, c1, r, bl, wc, bc, o_ref):
    cnt = jnp.maximum(c0[...] + c1[...], 1.0)
    hh = jnp.maximum((a0[...] + a1[...]) / cnt + bl[...] + r[...], 0.0)
    o_ref[...] = jnp.dot(hh, wc[...], preferred_element_type=jnp.float32) + bc[...]


def _fin(a0, a1, c0, c1, r, bl, wc, bc, bn=2000):
    n, h = a0.shape
    row = pl.BlockSpec((bn, h), lambda i: (i, 0))
    one = pl.BlockSpec((bn, 1), lambda i: (i, 0))
    return pl.pallas_call(
        _fin_body,
        grid=(n // bn,),
        in_specs=[row, row, one, one, row,
                  pl.BlockSpec((1, h), lambda i: (0, 0)),
                  pl.BlockSpec((h, 1), lambda i: (0, 0)),
                  pl.BlockSpec((1, 1), lambda i: (0, 0))],
        out_specs=pl.BlockSpec((bn, 1), lambda i: (i, 0)),
        out_shape=jax.ShapeDtypeStruct((n, 1), jnp.float32),
    )(a0, a1, c0, c1, r, bl, wc, bc)


def kernel(x, edge_index, Wl1, bl1, Wr1, Wl2, bl2, Wr2, Wc, bc):
    n, d = x.shape
    h = Wl1.shape[1]
    e = edge_index.shape[1]
    epw = e // NW
    c = -(-(-(-epw // K)) // NB) * NB
    pad = c * K - epw
    na, npt, _, _ = _acc_geom(n)

    src = edge_index[0]
    if pad:
        src = jnp.pad(src.reshape(NW, epw), ((0, 0), (0, pad)))
    src = src.reshape(NW, c, K)
    if pad:
        padvals = n + (jnp.arange(pad, dtype=jnp.int32) % PADR)
        padvals = jnp.broadcast_to(padvals, (NW, pad))
        dst = jnp.concatenate([edge_index[1].reshape(NW, epw), padvals], axis=1)
    else:
        dst = edge_index[1]
    dst = dst.reshape(NW, c, K)
    zeros2 = jnp.zeros((npt, h), jnp.float32)
    zeros1 = jnp.zeros((na,), jnp.float32)
    ones = jnp.ones((K,), jnp.float32)

    p1, r1 = _proj2(x, Wl1, Wr1)
    a0, a1, c0, c1 = _make_agg(n, h, e, True)(p1, src, dst, zeros2, zeros1, ones)
    c0 = c0.reshape(n, 1)
    c1 = c1.reshape(n, 1)
    p2, r2 = _comb(a0, a1, c0, c1, r1, bl1.reshape(1, h), Wl2, Wr2)
    b0, b1 = _make_agg(n, h, e, False)(p2, src, dst, zeros2)
    return _fin(b0, b1, c0, c1, r2, bl2.reshape(1, h), Wc, bc.reshape(1, 1))
